# Initial kernel scaffold; baseline (speedup 1.0000x reference)
import functools

import jax
import jax.numpy as jnp
from jax import lax
from jax.experimental import pallas as pl
from jax.experimental.pallas import tpu as pltpu
from jax.experimental.pallas import tpu_sc as plsc

NC = 2
NS = 16
NW = NC * NS

CHUNK = 1024


@functools.cache
def _build(V, D, B):
  assert B % NW == 0
  b_per_w = B // NW
  assert b_per_w % CHUNK == 0
  n_chunks = b_per_w // CHUNK
  mesh = plsc.VectorSubcoreMesh(
      core_axis_name="c", subcore_axis_name="s", num_cores=NC, num_subcores=NS)

  @functools.partial(
      pl.kernel,
      out_type=jax.ShapeDtypeStruct((B, D), jnp.float32),
      mesh=mesh,
      scratch_types=[
          pltpu.VMEM((CHUNK,), jnp.int32),
          pltpu.VMEM((CHUNK, D), jnp.float32),
          pltpu.SemaphoreType.DMA,
      ],
  )
  def gather_kernel(table_hbm, idx_hbm, out_hbm, idx_v, rows_v, sem):
    wid = lax.axis_index("s") * NC + lax.axis_index("c")
    base = wid * b_per_w

    def step(i, carry):
      cbase = base + i * CHUNK
      pltpu.sync_copy(idx_hbm.at[pl.ds(cbase, CHUNK)], idx_v)
      pltpu.async_copy(table_hbm.at[idx_v], rows_v, sem).wait()
      pltpu.sync_copy(rows_v, out_hbm.at[pl.ds(cbase, CHUNK)])
      return carry

    lax.fori_loop(0, n_chunks, step, 0)

  return gather_kernel


def kernel(data, ivectors):
  B0, B1 = data.shape
  V, D = ivectors.shape
  idx = data.reshape(-1).astype(jnp.int32)
  out = _build(V, D, B0 * B1)(ivectors, idx)
  return out.reshape(B0, B1, D)


# speedup vs baseline: 1.8443x; 1.8443x over previous
import functools

import jax
import jax.numpy as jnp
from jax import lax
from jax.experimental import pallas as pl
from jax.experimental.pallas import tpu as pltpu
from jax.experimental.pallas import tpu_sc as plsc

NC = 2
NS = 16
NW = NC * NS

CHUNK = 1024


@functools.cache
def _build(V, D, B):
  assert B % NW == 0
  b_per_w = B // NW
  assert b_per_w % CHUNK == 0
  n_chunks = b_per_w // CHUNK
  mesh = plsc.VectorSubcoreMesh(
      core_axis_name="c", subcore_axis_name="s", num_cores=NC, num_subcores=NS)

  @functools.partial(
      pl.kernel,
      out_type=jax.ShapeDtypeStruct((B, D), jnp.float32),
      mesh=mesh,
      scratch_types=[
          pltpu.VMEM((CHUNK,), jnp.int32),
          pltpu.VMEM((CHUNK, D), jnp.float32),
          pltpu.SemaphoreType.DMA,
      ],
      compiler_params=pltpu.CompilerParams(use_tc_tiling_on_sc=False),
  )
  def gather_kernel(table_hbm, idx_hbm, out_hbm, idx_v, rows_v, sem):
    wid = lax.axis_index("s") * NC + lax.axis_index("c")
    base = wid * b_per_w

    def step(i, carry):
      cbase = base + i * CHUNK
      pltpu.sync_copy(idx_hbm.at[pl.ds(cbase, CHUNK)], idx_v)
      pltpu.async_copy(table_hbm.at[idx_v], rows_v, sem).wait()
      pltpu.sync_copy(rows_v, out_hbm.at[pl.ds(cbase, CHUNK)])
      return carry

    lax.fori_loop(0, n_chunks, step, 0)

  return gather_kernel


def kernel(data, ivectors):
  B0, B1 = data.shape
  V, D = ivectors.shape
  idx = data.reshape(-1).astype(jnp.int32)
  out = _build(V, D, B0 * B1)(ivectors, idx)
  return out.reshape(B0, B1, D)


# speedup vs baseline: 1.8751x; 1.0167x over previous
import functools

import jax
import jax.numpy as jnp
from jax import lax
from jax.experimental import pallas as pl
from jax.experimental.pallas import tpu as pltpu
from jax.experimental.pallas import tpu_sc as plsc

NC = 2
NS = 16
NW = NC * NS

CHUNK = 800
NBUF = 2


@functools.cache
def _build(V, D, B):
  assert B % NW == 0
  b_per_w = B // NW
  assert b_per_w % (CHUNK * NBUF) == 0
  n_chunks = b_per_w // CHUNK
  mesh = plsc.VectorSubcoreMesh(
      core_axis_name="c", subcore_axis_name="s", num_cores=NC, num_subcores=NS)

  @functools.partial(
      pl.kernel,
      out_type=jax.ShapeDtypeStruct((B, D), jnp.float32),
      mesh=mesh,
      scratch_types=[
          pltpu.VMEM((b_per_w,), jnp.int32),
          [pltpu.VMEM((CHUNK, D), jnp.float32) for _ in range(NBUF)],
          [pltpu.SemaphoreType.DMA for _ in range(NBUF)],
          [pltpu.SemaphoreType.DMA for _ in range(NBUF)],
      ],
      compiler_params=pltpu.CompilerParams(use_tc_tiling_on_sc=False),
  )
  def gather_kernel(table_hbm, idx_hbm, out_hbm, idx_v, rows, gsem, ssem):
    wid = lax.axis_index("s") * NC + lax.axis_index("c")
    base = wid * b_per_w
    pltpu.sync_copy(idx_hbm.at[pl.ds(base, b_per_w)], idx_v)

    def start_gather(c, b):
      pltpu.async_copy(
          table_hbm.at[idx_v.at[pl.ds(c * CHUNK, CHUNK)]], rows[b], gsem[b])

    for b in range(NBUF):
      start_gather(b, b)

    def step(p, carry):
      c0 = p * NBUF
      for b in range(NBUF):
        c = c0 + b
        pltpu.make_async_copy(
            table_hbm.at[idx_v.at[pl.ds(c * CHUNK, CHUNK)]], rows[b],
            gsem[b]).wait()
        pltpu.async_copy(
            rows[b], out_hbm.at[pl.ds(base + c * CHUNK, CHUNK)], ssem[b])

        @pl.when(c + NBUF < n_chunks)
        def _():
          pltpu.make_async_copy(
              rows[b], out_hbm.at[pl.ds(base + c * CHUNK, CHUNK)],
              ssem[b]).wait()
          start_gather(c + NBUF, b)

      return carry

    lax.fori_loop(0, n_chunks // NBUF, step, 0)
    for b in range(NBUF):
      c = n_chunks - NBUF + b
      pltpu.make_async_copy(
          rows[b], out_hbm.at[pl.ds(base + c * CHUNK, CHUNK)], ssem[b]).wait()

  return gather_kernel


def kernel(data, ivectors):
  B0, B1 = data.shape
  V, D = ivectors.shape
  idx = data.reshape(-1).astype(jnp.int32)
  out = _build(V, D, B0 * B1)(ivectors, idx)
  return out.reshape(B0, B1, D)
